# Initial kernel scaffold; baseline (speedup 1.0000x reference)
import jax
import jax.numpy as jnp
from jax.experimental import pallas as pl


def kernel(user_emb, item_emb, v_feat, t_feat, W_ir, b_ir, W_it, b_it, W_tr, b_tr, W_tt, b_tt, W_sc1, b_sc1, w_sc2, W_ib, b_ib, W_tb, b_tb, ui_edges, ii_image_edges, ii_text_edges):
    raise NotImplementedError("write your pallas kernel here")


# speedup vs baseline: 15.0347x; 15.0347x over previous
import jax
import jax.numpy as jnp
from jax import lax
from jax.experimental import pallas as pl
from jax.experimental.pallas import tpu as pltpu
from jax.experimental.pallas import tpu_sc as plsc

_U = 50000
_I = 50000
_D = 64
_EUI = 800000
_EII = 500000

_UP = 51200
_IP = 51200
_NP = _UP + _IP

_CH = 1000
_ZR = 160
_CHD = 4000
_NSC = 2
_NT = 16


def _mesh():
  return plsc.VectorSubcoreMesh(
      core_axis_name="c", subcore_axis_name="s", num_cores=_NSC,
      num_subcores=_NT)


def _sc_params():
  return pltpu.CompilerParams(use_tc_tiling_on_sc=False)


def _fill(ref, rows, value):
  if len(ref.shape) == 2:
    def body(i, _):
      ref[i, :] = jnp.full((16,), value, jnp.float32)
      return _
    lax.fori_loop(0, rows, body, None)
  else:
    def body(i, _):
      ref[pl.ds(i * 16, 16)] = jnp.full((16,), value, jnp.float32)
      return _
    lax.fori_loop(0, rows, body, None)


def _make_segsum(n_dst, groups):
  stripe = n_dst // _NT
  assert stripe % _ZR == 0
  max_tabs = max(t for _, t in groups)
  n_out = sum(t for _, t in groups)

  out_type = [jax.ShapeDtypeStruct((8, n_dst, 16), jnp.float32)
              for _ in range(n_out)]
  scratch = ([pltpu.VMEM_SHARED((n_dst, 16), jnp.float32)] * max_tabs
             + [pltpu.VMEM((_CH,), jnp.int32)] * 2
             + [pltpu.VMEM((_CH, 16), jnp.float32),
                pltpu.VMEM((_ZR, 16), jnp.float32)])

  def body(*refs):
    pos = 0
    g_ins = []
    for e, ntab in groups:
      tabs = [refs[pos + 2 + 4 * a:pos + 2 + 4 * (a + 1)]
              for a in range(ntab)]
      g_ins.append((refs[pos], refs[pos + 1], tabs))
      pos += 2 + 4 * ntab
    outs = refs[pos:pos + n_out]
    accs = refs[pos + n_out:pos + n_out + max_tabs]
    r_v, c_v, rows_v, zbuf = refs[pos + n_out + max_tabs:]

    sc = lax.axis_index("c")
    t = lax.axis_index("s")
    _fill(zbuf, _ZR, 0.0)

    out_i = 0
    for g, (e, ntab) in enumerate(groups):
      r_hbm, c_hbm, tabs = g_ins[g]
      nch_sc = e // _CH // _NSC
      n_my = (nch_sc - t + _NT - 1) // _NT
      for k in range(4):
        for a in range(ntab):
          for zz in range(stripe // _ZR):
            pltpu.sync_copy(
                zbuf, accs[a].at[pl.ds(t * stripe + zz * _ZR, _ZR)])
        plsc.subcore_barrier()

        def chunk_body(i, _, r_hbm=r_hbm, c_hbm=c_hbm, tabs=tabs,
                       nch_sc=nch_sc, ntab=ntab, k=k):
          j = i * _NT + t
          base = (sc * nch_sc + j) * _CH
          pltpu.sync_copy(r_hbm.at[pl.ds(base, _CH)], r_v)
          pltpu.sync_copy(c_hbm.at[pl.ds(base, _CH)], c_v)
          for a in range(ntab):
            pltpu.sync_copy(tabs[a][k].at[c_v], rows_v)
            pltpu.sync_copy(rows_v, accs[a].at[r_v], add=True)
          return _

        lax.fori_loop(0, n_my, chunk_body, None)
        plsc.subcore_barrier()
        for a in range(ntab):
          for off in range(0, stripe, _CH):
            sz = min(_CH, stripe - off)
            pltpu.sync_copy(accs[a].at[pl.ds(t * stripe + off, sz)],
                            rows_v.at[pl.ds(0, sz)])
            pltpu.sync_copy(
                rows_v.at[pl.ds(0, sz)],
                outs[out_i + a].at[k * 2 + sc, pl.ds(t * stripe + off, sz)])
        plsc.subcore_barrier()
      out_i += ntab

  return pl.kernel(body, out_type=out_type, mesh=_mesh(),
                   scratch_types=scratch, compiler_params=_sc_params())


def _deg_kernel(src, dst, img_r, txt_r):
  n_z = 25
  out_type = [jax.ShapeDtypeStruct((_U,), jnp.float32) for _ in range(4)]
  scratch = [pltpu.VMEM_SHARED((_U,), jnp.float32)] * 2 + [
      pltpu.VMEM((_CHD,), jnp.int32),
      pltpu.VMEM((_CHD,), jnp.float32),
      pltpu.VMEM((2000,), jnp.float32),
  ]

  def body(src_r, dst_r, imr, txr, o_du, o_di, o_dim, o_dtx,
           acc0, acc1, idx_v, ones_v, zb):
    sc = lax.axis_index("c")
    t = lax.axis_index("s")
    _fill(ones_v, _CHD // 16, 1.0)
    _fill(zb, 125, 0.0)

    n_zmy = (n_z - t + _NT - 1) // _NT

    def zero_body(i, _):
      j = i * _NT + t
      pltpu.sync_copy(zb, acc0.at[pl.ds(j * 2000, 2000)])
      pltpu.sync_copy(zb, acc1.at[pl.ds(j * 2000, 2000)])
      return _

    lax.fori_loop(0, n_zmy, zero_body, None)
    plsc.subcore_barrier()

    def count(arr, acc, e):
      nch = e // _CHD
      n_my = (nch - t + _NT - 1) // _NT

      def cb(i, _):
        j = i * _NT + t
        pltpu.sync_copy(arr.at[pl.ds(j * _CHD, _CHD)], idx_v)
        pltpu.sync_copy(ones_v, acc.at[idx_v], add=True)
        return _

      lax.fori_loop(0, n_my, cb, None)

    @pl.when(sc == 0)
    def _():
      count(src_r, acc0, _EUI)
      count(dst_r, acc1, _EUI)

    @pl.when(sc == 1)
    def _():
      count(imr, acc0, _EII)
      count(txr, acc1, _EII)

    plsc.subcore_barrier()

    def wb(acc, out):
      def wbody(i, _):
        j = i * _NT + t
        pltpu.sync_copy(acc.at[pl.ds(j * 2000, 2000)], zb)
        pltpu.sync_copy(zb, out.at[pl.ds(j * 2000, 2000)])
        return _
      lax.fori_loop(0, n_zmy, wbody, None)

    @pl.when(sc == 0)
    def _():
      wb(acc0, o_du)
      wb(acc1, o_di)

    @pl.when(sc == 1)
    def _():
      wb(acc0, o_dim)
      wb(acc1, o_dtx)

  fn = pl.kernel(body, out_type=out_type, mesh=_mesh(),
                 scratch_types=scratch, compiler_params=_sc_params())
  return fn(src, dst, img_r, txt_r)


_B = 1024
_BP = _B // 8


def _zsum_packed(z_ref):
  return [z_ref[2 * k] + z_ref[2 * k + 1] for k in range(4)]


def _pblock(i):
  return (i, 0)


def _full(shape):
  return pl.BlockSpec(shape, lambda i: tuple(0 for _ in shape))


def _pack16(v):
  return jnp.repeat(v.reshape(-1, 8), 16, axis=1)


def _scale_packed(chunks, sp, power, m):
  mp = m // 8

  def body(c0, c1, c2, c3, s_ref, *o_refs):
    sv = s_ref[...]
    sp_v = sv
    for _ in range(power - 1):
      sp_v = sp_v * sv
    for o, c in zip(o_refs, (c0, c1, c2, c3)):
      o[...] = sp_v * c[...]

  return pl.pallas_call(
      body,
      grid=(m // _B,),
      in_specs=[pl.BlockSpec((_BP, 128), _pblock)] * 5,
      out_specs=[pl.BlockSpec((_BP, 128), _pblock)] * 4,
      out_shape=[jax.ShapeDtypeStruct((mp, 128), jnp.float32)] * 4,
  )(*chunks, sp)


def _scale_z(z, sp, powers, m):
  mp = m // 8

  def body(z_ref, s_ref, *o_refs):
    zs = _zsum_packed(z_ref)
    sv = s_ref[...]
    oi = 0
    for p in powers:
      sp_v = sv
      for _ in range(p - 1):
        sp_v = sp_v * sv
      for k in range(4):
        o_refs[oi][...] = sp_v * zs[k]
        oi += 1

  return pl.pallas_call(
      body,
      grid=(m // _B,),
      in_specs=[pl.BlockSpec((8, _BP, 128), lambda i: (0, i, 0)),
                pl.BlockSpec((_BP, 128), _pblock)],
      out_specs=[pl.BlockSpec((_BP, 128), _pblock)] * (4 * len(powers)),
      out_shape=[jax.ShapeDtypeStruct((mp, 128), jnp.float32)]
      * (4 * len(powers)),
  )(z, sp)


def _feats_kernel(v_feat, t_feat, W_ir, b_ir, W_it, b_it, W_tr, b_tr,
                  W_tt, b_tt):
  def body(v_ref, t_ref, wir, bir, wit, bit, wtr, btr, wtt, btt, oi, ot):
    f = jnp.dot(v_ref[...], wir[...], preferred_element_type=jnp.float32)
    f = jnp.dot(f + bir[...], wit[...], preferred_element_type=jnp.float32)
    oi[...] = jax.nn.sigmoid(f + bit[...])
    g = jnp.dot(t_ref[...], wtr[...], preferred_element_type=jnp.float32)
    g = jnp.dot(g + btr[...], wtt[...], preferred_element_type=jnp.float32)
    ot[...] = jax.nn.sigmoid(g + btt[...])

  return pl.pallas_call(
      body,
      grid=(_IP // _B,),
      in_specs=[pl.BlockSpec((_B, 128), lambda i: (i, 0)),
                pl.BlockSpec((_B, 128), lambda i: (i, 0)),
                _full((128, _D)), _full((1, _D)), _full((_D, _D)),
                _full((1, _D)),
                _full((128, _D)), _full((1, _D)), _full((_D, _D)),
                _full((1, _D))],
      out_specs=[pl.BlockSpec((_B, _D), lambda i: (i, 0))] * 2,
      out_shape=[jax.ShapeDtypeStruct((_IP, _D), jnp.float32)] * 2,
  )(v_feat, t_feat, W_ir, b_ir.reshape(1, -1), W_it, b_it.reshape(1, -1),
    W_tr, b_tr.reshape(1, -1), W_tt, b_tt.reshape(1, -1))


def _final_kernel(user_emb, item_emb, ego1, ego2, img_e_arr, txt_e_arr,
                  W_sc1, b_sc1, w_sc2, W_ib, b_ib, W_tb, b_tb):
  nu = _UP // _B

  def body(ue, ie, e1, e2, ime, txe,
           wsc1, bsc1, wsc2, wib, bib, wtb, btb, o_ref):
    i = pl.program_id(0)
    is_user = i < nu
    ego0 = jnp.where(is_user, ue[...], ie[...])
    content = (ego0 + e1[...] + e2[...]) * (1.0 / 3.0)

    img_e = ime[...]
    txt_e = txe[...]

    si = jnp.dot(jnp.tanh(
        jnp.dot(img_e, wsc1[...], preferred_element_type=jnp.float32)
        + bsc1[...]), wsc2[...], preferred_element_type=jnp.float32)
    st = jnp.dot(jnp.tanh(
        jnp.dot(txt_e, wsc1[...], preferred_element_type=jnp.float32)
        + bsc1[...]), wsc2[...], preferred_element_type=jnp.float32)
    ai = jax.nn.sigmoid(si - st)
    fused = ai * img_e + (1.0 - ai) * txt_e

    sep_i = jax.nn.sigmoid(
        jnp.dot(content, wib[...], preferred_element_type=jnp.float32)
        + bib[...]) * img_e
    sep_t = jax.nn.sigmoid(
        jnp.dot(content, wtb[...], preferred_element_type=jnp.float32)
        + btb[...]) * txt_e
    o_ref[...] = content + fused + sep_i + sep_t

  umap = lambda i: (jnp.minimum(i, nu - 1), 0)
  imap = lambda i: (jnp.maximum(i - nu, 0), 0)
  nspec = pl.BlockSpec((_B, _D), lambda i: (i, 0))
  return pl.pallas_call(
      body,
      grid=(_NP // _B,),
      in_specs=[pl.BlockSpec((_B, _D), umap),
                pl.BlockSpec((_B, _D), imap),
                nspec, nspec, nspec, nspec,
                _full((_D, _D)), _full((1, _D)), _full((_D, 1)),
                _full((_D, _D)), _full((1, _D)),
                _full((_D, _D)), _full((1, _D))],
      out_specs=nspec,
      out_shape=jax.ShapeDtypeStruct((_NP, _D), jnp.float32),
  )(user_emb, item_emb, ego1, ego2, img_e_arr, txt_e_arr,
    W_sc1, b_sc1.reshape(1, -1), w_sc2, W_ib, b_ib.reshape(1, -1),
    W_tb, b_tb.reshape(1, -1))


def _sc_view(packed, m):
  return packed.reshape(m, 16)


def _packed_z(z, m):
  return z.reshape(8, m // 8, 128)


def _pack_glue(x, m):
  return [x[:, 16 * k:16 * (k + 1)].reshape(m // 8, 128) for k in range(4)]


def _unpack_glue(chunks, m):
  return jnp.concatenate([c.reshape(m, 16) for c in chunks], axis=1)


def _pad_rows(x, rows):
  return jnp.pad(x, ((0, rows - x.shape[0]), (0, 0)))


def _pad_vec(v, n):
  return jnp.pad(v, (0, n - v.shape[0]))


def kernel(user_emb, item_emb, v_feat, t_feat, W_ir, b_ir, W_it, b_it,
           W_tr, b_tr, W_tt, b_tt, W_sc1, b_sc1, w_sc2, W_ib, b_ib,
           W_tb, b_tb, ui_edges, ii_image_edges, ii_text_edges):
  src = ui_edges[0]
  dst = ui_edges[1]

  deg_u, deg_i, d_img, d_txt = _deg_kernel(
      src, dst, ii_image_edges[0], ii_text_edges[0])

  dinv_u = jnp.where(deg_u > 0, lax.rsqrt(deg_u), 0.0)
  dinv_i = jnp.where(deg_i > 0, lax.rsqrt(deg_i), 0.0)
  dp = _pack16(jnp.concatenate([_pad_vec(dinv_u, _UP),
                                _pad_vec(dinv_i, _IP)]))
  dup = _pack16(_pad_vec(jnp.where(deg_u > 0, 1.0 / deg_u, 0.0), _UP))
  dip = _pack16(_pad_vec(jnp.where(d_img > 0, lax.rsqrt(d_img), 0.0), _IP))
  dtp = _pack16(_pad_vec(jnp.where(d_txt > 0, lax.rsqrt(d_txt), 0.0), _IP))

  ue_p = _pad_rows(user_emb, _UP)
  ie_p = _pad_rows(item_emb, _IP)
  vf_p = _pad_rows(v_feat, _IP)
  tf_p = _pad_rows(t_feat, _IP)

  rows = jnp.concatenate([src, dst + _UP])
  cols = jnp.concatenate([dst + _UP, src])

  ue_chunks = _pack_glue(ue_p, _UP)
  ie_chunks = _pack_glue(ie_p, _IP)
  ego0p = [jnp.concatenate([u, i], axis=0)
           for u, i in zip(ue_chunks, ie_chunks)]
  y0p = _scale_packed(ego0p, dp, 1, _NP)

  ui_segsum = _make_segsum(_NP, [(2 * _EUI, 1)])
  (z1,) = ui_segsum(rows, cols, *[_sc_view(y, _NP) for y in y0p])
  sc1 = _scale_z(_packed_z(z1, _NP), dp, (1, 2), _NP)
  ego1p, y1p = sc1[:4], sc1[4:]
  (z2,) = ui_segsum(rows, cols, *[_sc_view(y, _NP) for y in y1p])

  f_img, f_txt = _feats_kernel(vf_p, tf_p, W_ir, b_ir, W_it, b_it,
                               W_tr, b_tr, W_tt, b_tt)
  yi0p = _scale_packed(_pack_glue(f_img, _IP), dip, 1, _IP)
  yt0p = _scale_packed(_pack_glue(f_txt, _IP), dtp, 1, _IP)

  ii_segsum = _make_segsum(_IP, [(_EII, 1), (_EII, 1)])
  zi1, zt1 = ii_segsum(
      ii_image_edges[0], ii_image_edges[1],
      *[_sc_view(y, _IP) for y in yi0p],
      ii_text_edges[0], ii_text_edges[1],
      *[_sc_view(y, _IP) for y in yt0p])
  yi1p = _scale_z(_packed_z(zi1, _IP), dip, (2,), _IP)
  yt1p = _scale_z(_packed_z(zt1, _IP), dtp, (2,), _IP)
  zi2, zt2 = ii_segsum(
      ii_image_edges[0], ii_image_edges[1],
      *[_sc_view(y, _IP) for y in yi1p],
      ii_text_edges[0], ii_text_edges[1],
      *[_sc_view(y, _IP) for y in yt1p])
  img_itp = _scale_z(_packed_z(zi2, _IP), dip, (1,), _IP)
  txt_itp = _scale_z(_packed_z(zt2, _IP), dtp, (1,), _IP)

  uagg = _make_segsum(_UP, [(_EUI, 2)])
  zu_i, zu_t = uagg(src, dst,
                    *[_sc_view(y, _IP) for y in img_itp],
                    *[_sc_view(y, _IP) for y in txt_itp])
  imgu_p = _scale_z(_packed_z(zu_i, _UP), dup, (1,), _UP)
  txtu_p = _scale_z(_packed_z(zu_t, _UP), dup, (1,), _UP)

  ego1 = _unpack_glue(ego1p, _NP)
  ego2p = _scale_z(_packed_z(z2, _NP), dp, (1,), _NP)
  ego2 = _unpack_glue(ego2p, _NP)
  img_e = jnp.concatenate(
      [_unpack_glue(imgu_p, _UP), _unpack_glue(img_itp, _IP)], axis=0)
  txt_e = jnp.concatenate(
      [_unpack_glue(txtu_p, _UP), _unpack_glue(txt_itp, _IP)], axis=0)

  out_pad = _final_kernel(ue_p, ie_p, ego1, ego2, img_e, txt_e,
                          W_sc1, b_sc1, w_sc2, W_ib, b_ib, W_tb, b_tb)
  return jnp.concatenate([out_pad[:_U], out_pad[_UP:_UP + _I]], axis=0)


# speedup vs baseline: 19.4831x; 1.2959x over previous
import jax
import jax.numpy as jnp
from jax import lax
from jax.experimental import pallas as pl
from jax.experimental.pallas import tpu as pltpu
from jax.experimental.pallas import tpu_sc as plsc

_U = 50000
_I = 50000
_D = 64
_EUI = 800000
_EII = 500000

_UP = 51200
_IP = 51200
_NP = _UP + _IP

_ZR = 160
_CHD = 4000
_NSC = 2
_NT = 16


def _mesh():
  return plsc.VectorSubcoreMesh(
      core_axis_name="c", subcore_axis_name="s", num_cores=_NSC,
      num_subcores=_NT)


def _sc_params():
  return pltpu.CompilerParams(use_tc_tiling_on_sc=False)


def _fill(ref, rows, value):
  if len(ref.shape) == 2:
    def body(i, _):
      ref[i, :] = jnp.full((16,), value, jnp.float32)
      return _
    lax.fori_loop(0, rows, body, None)
  else:
    def body(i, _):
      ref[pl.ds(i * 16, 16)] = jnp.full((16,), value, jnp.float32)
      return _
    lax.fori_loop(0, rows, body, None)


def _make_segsum(n_dst, groups, ch):
  stripe = n_dst // _NT
  assert stripe % _ZR == 0
  max_tabs = max(t for _, t in groups)
  n_out = sum(t for _, t in groups)
  n_slots = 4 if max_tabs == 1 else 2

  out_type = [jax.ShapeDtypeStruct((8, n_dst, 16), jnp.float32)
              for _ in range(n_out)]
  scratch = ([pltpu.VMEM_SHARED((n_dst, 16), jnp.float32)] * max_tabs
             + [pltpu.VMEM((ch,), jnp.int32)] * (2 * n_slots)
             + [pltpu.VMEM((ch, 16), jnp.float32)] * 2
             + [pltpu.VMEM((_ZR, 16), jnp.float32)]
             + [pltpu.SemaphoreType.DMA] * (n_slots + 4))

  def body(*refs):
    pos = 0
    g_ins = []
    for e, ntab in groups:
      tabs = [refs[pos + 2 + 4 * a:pos + 2 + 4 * (a + 1)]
              for a in range(ntab)]
      g_ins.append((refs[pos], refs[pos + 1], tabs))
      pos += 2 + 4 * ntab
    outs = refs[pos:pos + n_out]
    pos += n_out
    accs = refs[pos:pos + max_tabs]
    pos += max_tabs
    r_v = refs[pos:pos + n_slots]
    c_v = refs[pos + n_slots:pos + 2 * n_slots]
    pos += 2 * n_slots
    rows = refs[pos:pos + 2]
    zbuf = refs[pos + 2]
    sems = refs[pos + 3:]
    sem_i = sems[:n_slots]
    sem_g = sems[n_slots:n_slots + 2]
    sem_s = sems[n_slots + 2:n_slots + 4]

    sc = lax.axis_index("c")
    t = lax.axis_index("s")
    _fill(zbuf, _ZR, 0.0)

    out_i = 0
    for g, (e, ntab) in enumerate(groups):
      r_hbm, c_hbm, tabs = g_ins[g]
      nch_sc = e // ch // _NSC
      n_my = (nch_sc - t + _NT - 1) // _NT

      def base(cloc, nch_sc=nch_sc):
        return (sc * nch_sc + (cloc * _NT + t)) * ch

      def idx_start(cloc, s, r_hbm=r_hbm, c_hbm=c_hbm):
        b = base(cloc)
        pltpu.async_copy(r_hbm.at[pl.ds(b, ch)], r_v[s], sem_i[s])
        pltpu.async_copy(c_hbm.at[pl.ds(b, ch)], c_v[s], sem_i[s])

      def idx_wait(cloc, s, r_hbm=r_hbm, c_hbm=c_hbm):
        b = base(cloc)
        pltpu.make_async_copy(r_hbm.at[pl.ds(b, ch)], r_v[s],
                              sem_i[s]).wait()
        pltpu.make_async_copy(c_hbm.at[pl.ds(b, ch)], c_v[s],
                              sem_i[s]).wait()

      for k in range(4):
        tab_k = [tabs[a][k] for a in range(ntab)]
        acc_k = accs[:ntab]

        def g_start(s, b, a, tab_k=tab_k):
          pltpu.async_copy(tab_k[a].at[c_v[s]], rows[b], sem_g[b])

        def g_wait(s, b, a, tab_k=tab_k):
          pltpu.make_async_copy(tab_k[a].at[c_v[s]], rows[b],
                                sem_g[b]).wait()

        def s_start(s, b, a, acc_k=acc_k):
          pltpu.async_copy(rows[b], acc_k[a].at[r_v[s]], sem_s[b],
                           add=True)

        def s_wait(s, b, a, acc_k=acc_k):
          pltpu.make_async_copy(rows[b], acc_k[a].at[r_v[s]],
                                sem_s[b]).wait()

        for a in range(ntab):
          for zz in range(stripe // _ZR):
            pltpu.sync_copy(
                zbuf, accs[a].at[pl.ds(t * stripe + zz * _ZR, _ZR)])
        plsc.subcore_barrier()

        if ntab == 1:
          @pl.when(n_my >= 1)
          def _():
            idx_start(0, 0)

          @pl.when(n_my >= 2)
          def _():
            idx_start(1, 1)

          def pair(i2, s0, s1, n_my=n_my):
            c0 = 2 * i2
            c1 = c0 + 1
            o0 = (s0 + 2) % 4
            o1 = (s1 + 2) % 4
            v1 = c1 < n_my

            @pl.when(i2 >= 1)
            def _():
              s_wait(o0, 0, 0)

              @pl.when(c0 - 1 < n_my)
              def _():
                s_wait(o1, 1, 0)

            idx_wait(c0, s0)
            g_start(s0, 0, 0)

            @pl.when(v1)
            def _():
              idx_wait(c1, s1)
              g_start(s1, 1, 0)

            @pl.when(c0 + 2 < n_my)
            def _():
              idx_start(c0 + 2, o0)

            @pl.when(c1 + 2 < n_my)
            def _():
              idx_start(c1 + 2, o1)

            g_wait(s0, 0, 0)
            s_start(s0, 0, 0)

            @pl.when(v1)
            def _():
              g_wait(s1, 1, 0)
              s_start(s1, 1, 0)

          def pair_body(i2, _):
            @pl.when(i2 % 2 == 0)
            def _():
              pair(i2, 0, 1)

            @pl.when(i2 % 2 == 1)
            def _():
              pair(i2, 2, 3)
            return _

          n_pairs = (n_my + 1) // 2
          lax.fori_loop(0, n_pairs, pair_body, None)
          last = n_pairs - 1

          def drain(s0, s1, n_my=n_my, last=last):
            s_wait(s0, 0, 0)

            @pl.when(n_my % 2 == 0)
            def _():
              s_wait(s1, 1, 0)

          @pl.when((n_my >= 1) & (last % 2 == 0))
          def _():
            drain(0, 1)

          @pl.when((n_my >= 1) & (last % 2 == 1))
          def _():
            drain(2, 3)

        else:
          @pl.when(n_my >= 1)
          def _():
            idx_start(0, 0)

          def single(c, s, o, n_my=n_my):
            @pl.when(c >= 1)
            def _():
              s_wait(o, 0, 0)
              s_wait(o, 1, 1)

            idx_wait(c, s)
            g_start(s, 0, 0)
            g_start(s, 1, 1)

            @pl.when(c + 1 < n_my)
            def _():
              idx_start(c + 1, o)

            g_wait(s, 0, 0)
            s_start(s, 0, 0)
            g_wait(s, 1, 1)
            s_start(s, 1, 1)

          def single_body(c, _):
            @pl.when(c % 2 == 0)
            def _():
              single(c, 0, 1)

            @pl.when(c % 2 == 1)
            def _():
              single(c, 1, 0)
            return _

          lax.fori_loop(0, n_my, single_body, None)
          last = n_my - 1

          @pl.when((n_my >= 1) & (last % 2 == 0))
          def _():
            s_wait(0, 0, 0)
            s_wait(0, 1, 1)

          @pl.when((n_my >= 1) & (last % 2 == 1))
          def _():
            s_wait(1, 0, 0)
            s_wait(1, 1, 1)

        plsc.subcore_barrier()
        for a in range(ntab):
          for off in range(0, stripe, ch):
            sz = min(ch, stripe - off)
            pltpu.sync_copy(accs[a].at[pl.ds(t * stripe + off, sz)],
                            rows[0].at[pl.ds(0, sz)])
            pltpu.sync_copy(
                rows[0].at[pl.ds(0, sz)],
                outs[out_i + a].at[k * 2 + sc, pl.ds(t * stripe + off, sz)])
        plsc.subcore_barrier()
      out_i += ntab

  return pl.kernel(body, out_type=out_type, mesh=_mesh(),
                   scratch_types=scratch, compiler_params=_sc_params())


def _deg_kernel(src, dst, img_r, txt_r):
  n_z = 25
  out_type = [jax.ShapeDtypeStruct((_U,), jnp.float32) for _ in range(4)]
  scratch = [pltpu.VMEM_SHARED((_U,), jnp.float32)] * 2 + [
      pltpu.VMEM((_CHD,), jnp.int32),
      pltpu.VMEM((_CHD,), jnp.float32),
      pltpu.VMEM((2000,), jnp.float32),
  ]

  def body(src_r, dst_r, imr, txr, o_du, o_di, o_dim, o_dtx,
           acc0, acc1, idx_v, ones_v, zb):
    sc = lax.axis_index("c")
    t = lax.axis_index("s")
    _fill(ones_v, _CHD // 16, 1.0)
    _fill(zb, 125, 0.0)

    n_zmy = (n_z - t + _NT - 1) // _NT

    def zero_body(i, _):
      j = i * _NT + t
      pltpu.sync_copy(zb, acc0.at[pl.ds(j * 2000, 2000)])
      pltpu.sync_copy(zb, acc1.at[pl.ds(j * 2000, 2000)])
      return _

    lax.fori_loop(0, n_zmy, zero_body, None)
    plsc.subcore_barrier()

    def count(arr, acc, e):
      nch = e // _CHD
      n_my = (nch - t + _NT - 1) // _NT

      def cb(i, _):
        j = i * _NT + t
        pltpu.sync_copy(arr.at[pl.ds(j * _CHD, _CHD)], idx_v)
        pltpu.sync_copy(ones_v, acc.at[idx_v], add=True)
        return _

      lax.fori_loop(0, n_my, cb, None)

    @pl.when(sc == 0)
    def _():
      count(src_r, acc0, _EUI)
      count(dst_r, acc1, _EUI)

    @pl.when(sc == 1)
    def _():
      count(imr, acc0, _EII)
      count(txr, acc1, _EII)

    plsc.subcore_barrier()

    def wb(acc, out):
      def wbody(i, _):
        j = i * _NT + t
        pltpu.sync_copy(acc.at[pl.ds(j * 2000, 2000)], zb)
        pltpu.sync_copy(zb, out.at[pl.ds(j * 2000, 2000)])
        return _
      lax.fori_loop(0, n_zmy, wbody, None)

    @pl.when(sc == 0)
    def _():
      wb(acc0, o_du)
      wb(acc1, o_di)

    @pl.when(sc == 1)
    def _():
      wb(acc0, o_dim)
      wb(acc1, o_dtx)

  fn = pl.kernel(body, out_type=out_type, mesh=_mesh(),
                 scratch_types=scratch, compiler_params=_sc_params())
  return fn(src, dst, img_r, txt_r)


_B = 1024
_BP = _B // 8


def _zsum_packed(z_ref):
  return [z_ref[2 * k] + z_ref[2 * k + 1] for k in range(4)]


def _pblock(i):
  return (i, 0)


def _full(shape):
  return pl.BlockSpec(shape, lambda i: tuple(0 for _ in shape))


def _pack16(v):
  return jnp.repeat(v.reshape(-1, 8), 16, axis=1)


def _scale_packed(chunks, sp, power, m):
  mp = m // 8

  def body(c0, c1, c2, c3, s_ref, *o_refs):
    sv = s_ref[...]
    sp_v = sv
    for _ in range(power - 1):
      sp_v = sp_v * sv
    for o, c in zip(o_refs, (c0, c1, c2, c3)):
      o[...] = sp_v * c[...]

  return pl.pallas_call(
      body,
      grid=(m // _B,),
      in_specs=[pl.BlockSpec((_BP, 128), _pblock)] * 5,
      out_specs=[pl.BlockSpec((_BP, 128), _pblock)] * 4,
      out_shape=[jax.ShapeDtypeStruct((mp, 128), jnp.float32)] * 4,
  )(*chunks, sp)


def _scale_z(z, sp, powers, m):
  mp = m // 8

  def body(z_ref, s_ref, *o_refs):
    zs = _zsum_packed(z_ref)
    sv = s_ref[...]
    oi = 0
    for p in powers:
      sp_v = sv
      for _ in range(p - 1):
        sp_v = sp_v * sv
      for k in range(4):
        o_refs[oi][...] = sp_v * zs[k]
        oi += 1

  return pl.pallas_call(
      body,
      grid=(m // _B,),
      in_specs=[pl.BlockSpec((8, _BP, 128), lambda i: (0, i, 0)),
                pl.BlockSpec((_BP, 128), _pblock)],
      out_specs=[pl.BlockSpec((_BP, 128), _pblock)] * (4 * len(powers)),
      out_shape=[jax.ShapeDtypeStruct((mp, 128), jnp.float32)]
      * (4 * len(powers)),
  )(z, sp)


def _feats_kernel(v_feat, t_feat, W_ir, b_ir, W_it, b_it, W_tr, b_tr,
                  W_tt, b_tt):
  def body(v_ref, t_ref, wir, bir, wit, bit, wtr, btr, wtt, btt, oi, ot):
    f = jnp.dot(v_ref[...], wir[...], preferred_element_type=jnp.float32)
    f = jnp.dot(f + bir[...], wit[...], preferred_element_type=jnp.float32)
    oi[...] = jax.nn.sigmoid(f + bit[...])
    g = jnp.dot(t_ref[...], wtr[...], preferred_element_type=jnp.float32)
    g = jnp.dot(g + btr[...], wtt[...], preferred_element_type=jnp.float32)
    ot[...] = jax.nn.sigmoid(g + btt[...])

  return pl.pallas_call(
      body,
      grid=(_IP // _B,),
      in_specs=[pl.BlockSpec((_B, 128), lambda i: (i, 0)),
                pl.BlockSpec((_B, 128), lambda i: (i, 0)),
                _full((128, _D)), _full((1, _D)), _full((_D, _D)),
                _full((1, _D)),
                _full((128, _D)), _full((1, _D)), _full((_D, _D)),
                _full((1, _D))],
      out_specs=[pl.BlockSpec((_B, _D), lambda i: (i, 0))] * 2,
      out_shape=[jax.ShapeDtypeStruct((_IP, _D), jnp.float32)] * 2,
  )(v_feat, t_feat, W_ir, b_ir.reshape(1, -1), W_it, b_it.reshape(1, -1),
    W_tr, b_tr.reshape(1, -1), W_tt, b_tt.reshape(1, -1))


def _final_kernel(user_emb, item_emb, ego1, ego2, img_e_arr, txt_e_arr,
                  W_sc1, b_sc1, w_sc2, W_ib, b_ib, W_tb, b_tb):
  nu = _UP // _B

  def body(ue, ie, e1, e2, ime, txe,
           wsc1, bsc1, wsc2, wib, bib, wtb, btb, o_ref):
    i = pl.program_id(0)
    is_user = i < nu
    ego0 = jnp.where(is_user, ue[...], ie[...])
    content = (ego0 + e1[...] + e2[...]) * (1.0 / 3.0)

    img_e = ime[...]
    txt_e = txe[...]

    si = jnp.dot(jnp.tanh(
        jnp.dot(img_e, wsc1[...], preferred_element_type=jnp.float32)
        + bsc1[...]), wsc2[...], preferred_element_type=jnp.float32)
    st = jnp.dot(jnp.tanh(
        jnp.dot(txt_e, wsc1[...], preferred_element_type=jnp.float32)
        + bsc1[...]), wsc2[...], preferred_element_type=jnp.float32)
    ai = jax.nn.sigmoid(si - st)
    fused = ai * img_e + (1.0 - ai) * txt_e

    sep_i = jax.nn.sigmoid(
        jnp.dot(content, wib[...], preferred_element_type=jnp.float32)
        + bib[...]) * img_e
    sep_t = jax.nn.sigmoid(
        jnp.dot(content, wtb[...], preferred_element_type=jnp.float32)
        + btb[...]) * txt_e
    o_ref[...] = content + fused + sep_i + sep_t

  umap = lambda i: (jnp.minimum(i, nu - 1), 0)
  imap = lambda i: (jnp.maximum(i - nu, 0), 0)
  nspec = pl.BlockSpec((_B, _D), lambda i: (i, 0))
  return pl.pallas_call(
      body,
      grid=(_NP // _B,),
      in_specs=[pl.BlockSpec((_B, _D), umap),
                pl.BlockSpec((_B, _D), imap),
                nspec, nspec, nspec, nspec,
                _full((_D, _D)), _full((1, _D)), _full((_D, 1)),
                _full((_D, _D)), _full((1, _D)),
                _full((_D, _D)), _full((1, _D))],
      out_specs=nspec,
      out_shape=jax.ShapeDtypeStruct((_NP, _D), jnp.float32),
  )(user_emb, item_emb, ego1, ego2, img_e_arr, txt_e_arr,
    W_sc1, b_sc1.reshape(1, -1), w_sc2, W_ib, b_ib.reshape(1, -1),
    W_tb, b_tb.reshape(1, -1))


def _sc_view(packed, m):
  return packed.reshape(m, 16)


def _packed_z(z, m):
  return z.reshape(8, m // 8, 128)


def _pack_glue(x, m):
  return [x[:, 16 * k:16 * (k + 1)].reshape(m // 8, 128) for k in range(4)]


def _unpack_glue(chunks, m):
  return jnp.concatenate([c.reshape(m, 16) for c in chunks], axis=1)


def _pad_rows(x, rows):
  return jnp.pad(x, ((0, rows - x.shape[0]), (0, 0)))


def _pad_vec(v, n):
  return jnp.pad(v, (0, n - v.shape[0]))


def kernel(user_emb, item_emb, v_feat, t_feat, W_ir, b_ir, W_it, b_it,
           W_tr, b_tr, W_tt, b_tt, W_sc1, b_sc1, w_sc2, W_ib, b_ib,
           W_tb, b_tb, ui_edges, ii_image_edges, ii_text_edges):
  src = ui_edges[0]
  dst = ui_edges[1]

  deg_u, deg_i, d_img, d_txt = _deg_kernel(
      src, dst, ii_image_edges[0], ii_text_edges[0])

  dinv_u = jnp.where(deg_u > 0, lax.rsqrt(deg_u), 0.0)
  dinv_i = jnp.where(deg_i > 0, lax.rsqrt(deg_i), 0.0)
  dp = _pack16(jnp.concatenate([_pad_vec(dinv_u, _UP),
                                _pad_vec(dinv_i, _IP)]))
  dup = _pack16(_pad_vec(jnp.where(deg_u > 0, 1.0 / deg_u, 0.0), _UP))
  dip = _pack16(_pad_vec(jnp.where(d_img > 0, lax.rsqrt(d_img), 0.0), _IP))
  dtp = _pack16(_pad_vec(jnp.where(d_txt > 0, lax.rsqrt(d_txt), 0.0), _IP))

  ue_p = _pad_rows(user_emb, _UP)
  ie_p = _pad_rows(item_emb, _IP)
  vf_p = _pad_rows(v_feat, _IP)
  tf_p = _pad_rows(t_feat, _IP)

  rows = jnp.concatenate([src, dst + _UP])
  cols = jnp.concatenate([dst + _UP, src])

  ue_chunks = _pack_glue(ue_p, _UP)
  ie_chunks = _pack_glue(ie_p, _IP)
  ego0p = [jnp.concatenate([u, i], axis=0)
           for u, i in zip(ue_chunks, ie_chunks)]
  y0p = _scale_packed(ego0p, dp, 1, _NP)

  ui_segsum = _make_segsum(_NP, [(2 * _EUI, 1)], 400)
  (z1,) = ui_segsum(rows, cols, *[_sc_view(y, _NP) for y in y0p])
  sc1 = _scale_z(_packed_z(z1, _NP), dp, (1, 2), _NP)
  ego1p, y1p = sc1[:4], sc1[4:]
  (z2,) = ui_segsum(rows, cols, *[_sc_view(y, _NP) for y in y1p])

  f_img, f_txt = _feats_kernel(vf_p, tf_p, W_ir, b_ir, W_it, b_it,
                               W_tr, b_tr, W_tt, b_tt)
  yi0p = _scale_packed(_pack_glue(f_img, _IP), dip, 1, _IP)
  yt0p = _scale_packed(_pack_glue(f_txt, _IP), dtp, 1, _IP)

  ii_segsum = _make_segsum(_IP, [(_EII, 1), (_EII, 1)], 1000)
  zi1, zt1 = ii_segsum(
      ii_image_edges[0], ii_image_edges[1],
      *[_sc_view(y, _IP) for y in yi0p],
      ii_text_edges[0], ii_text_edges[1],
      *[_sc_view(y, _IP) for y in yt0p])
  yi1p = _scale_z(_packed_z(zi1, _IP), dip, (2,), _IP)
  yt1p = _scale_z(_packed_z(zt1, _IP), dtp, (2,), _IP)
  zi2, zt2 = ii_segsum(
      ii_image_edges[0], ii_image_edges[1],
      *[_sc_view(y, _IP) for y in yi1p],
      ii_text_edges[0], ii_text_edges[1],
      *[_sc_view(y, _IP) for y in yt1p])
  img_itp = _scale_z(_packed_z(zi2, _IP), dip, (1,), _IP)
  txt_itp = _scale_z(_packed_z(zt2, _IP), dtp, (1,), _IP)

  uagg = _make_segsum(_UP, [(_EUI, 2)], 400)
  zu_i, zu_t = uagg(src, dst,
                    *[_sc_view(y, _IP) for y in img_itp],
                    *[_sc_view(y, _IP) for y in txt_itp])
  imgu_p = _scale_z(_packed_z(zu_i, _UP), dup, (1,), _UP)
  txtu_p = _scale_z(_packed_z(zu_t, _UP), dup, (1,), _UP)

  ego1 = _unpack_glue(ego1p, _NP)
  ego2p = _scale_z(_packed_z(z2, _NP), dp, (1,), _NP)
  ego2 = _unpack_glue(ego2p, _NP)
  img_e = jnp.concatenate(
      [_unpack_glue(imgu_p, _UP), _unpack_glue(img_itp, _IP)], axis=0)
  txt_e = jnp.concatenate(
      [_unpack_glue(txtu_p, _UP), _unpack_glue(txt_itp, _IP)], axis=0)

  out_pad = _final_kernel(ue_p, ie_p, ego1, ego2, img_e, txt_e,
                          W_sc1, b_sc1, w_sc2, W_ib, b_ib, W_tb, b_tb)
  return jnp.concatenate([out_pad[:_U], out_pad[_UP:_UP + _I]], axis=0)


# speedup vs baseline: 24.7407x; 1.2699x over previous
import jax
import jax.numpy as jnp
from jax import lax
from jax.experimental import pallas as pl
from jax.experimental.pallas import tpu as pltpu
from jax.experimental.pallas import tpu_sc as plsc

_U = 50000
_I = 50000
_D = 64
_EUI = 800000
_EII = 500000

_UP = 51200
_IP = 51200
_NP = _UP + _IP

_ZR = 160
_CHD = 4000
_NSC = 2
_NT = 16


def _mesh():
  return plsc.VectorSubcoreMesh(
      core_axis_name="c", subcore_axis_name="s", num_cores=_NSC,
      num_subcores=_NT)


def _sc_params():
  return pltpu.CompilerParams(use_tc_tiling_on_sc=False)


def _fill(ref, rows, value):
  if len(ref.shape) == 2:
    segs = ref.shape[1] // 16

    def body(i, _):
      for jj in range(segs):
        ref[i, pl.ds(jj * 16, 16)] = jnp.full((16,), value, jnp.float32)
      return _
    lax.fori_loop(0, rows, body, None)
  else:
    def body(i, _):
      ref[pl.ds(i * 16, 16)] = jnp.full((16,), value, jnp.float32)
      return _
    lax.fori_loop(0, rows, body, None)


def _make_segsum(n_dst, groups, ch, w=16):
  nk = 64 // w
  zr = 2560 // w
  stripe = n_dst // _NT
  assert stripe % zr == 0
  max_tabs = max(t for _, t in groups)
  n_out = sum(t for _, t in groups)
  n_slots = 4 if max_tabs == 1 else 2

  out_type = [jax.ShapeDtypeStruct((2 * nk, n_dst, w), jnp.float32)
              for _ in range(n_out)]
  scratch = ([pltpu.VMEM_SHARED((n_dst, w), jnp.float32)] * max_tabs
             + [pltpu.VMEM((ch,), jnp.int32)] * (2 * n_slots)
             + [pltpu.VMEM((ch, w), jnp.float32)] * 2
             + [pltpu.VMEM((zr, w), jnp.float32)]
             + [pltpu.SemaphoreType.DMA] * (n_slots + 4))

  def body(*refs):
    pos = 0
    g_ins = []
    for e, ntab in groups:
      tabs = [refs[pos + 2 + nk * a:pos + 2 + nk * (a + 1)]
              for a in range(ntab)]
      g_ins.append((refs[pos], refs[pos + 1], tabs))
      pos += 2 + nk * ntab
    outs = refs[pos:pos + n_out]
    pos += n_out
    accs = refs[pos:pos + max_tabs]
    pos += max_tabs
    r_v = refs[pos:pos + n_slots]
    c_v = refs[pos + n_slots:pos + 2 * n_slots]
    pos += 2 * n_slots
    rows = refs[pos:pos + 2]
    zbuf = refs[pos + 2]
    sems = refs[pos + 3:]
    sem_i = sems[:n_slots]
    sem_g = sems[n_slots:n_slots + 2]
    sem_s = sems[n_slots + 2:n_slots + 4]

    sc = lax.axis_index("c")
    t = lax.axis_index("s")
    _fill(zbuf, zr, 0.0)

    out_i = 0
    for g, (e, ntab) in enumerate(groups):
      r_hbm, c_hbm, tabs = g_ins[g]
      nch_sc = e // ch // _NSC
      n_my = (nch_sc - t + _NT - 1) // _NT

      def base(cloc, nch_sc=nch_sc):
        return (sc * nch_sc + (cloc * _NT + t)) * ch

      def idx_start(cloc, s, r_hbm=r_hbm, c_hbm=c_hbm):
        b = base(cloc)
        pltpu.async_copy(r_hbm.at[pl.ds(b, ch)], r_v[s], sem_i[s])
        pltpu.async_copy(c_hbm.at[pl.ds(b, ch)], c_v[s], sem_i[s])

      def idx_wait(cloc, s, r_hbm=r_hbm, c_hbm=c_hbm):
        b = base(cloc)
        pltpu.make_async_copy(r_hbm.at[pl.ds(b, ch)], r_v[s],
                              sem_i[s]).wait()
        pltpu.make_async_copy(c_hbm.at[pl.ds(b, ch)], c_v[s],
                              sem_i[s]).wait()

      for k in range(nk):
        tab_k = [tabs[a][k] for a in range(ntab)]
        acc_k = accs[:ntab]

        def g_start(s, b, a, tab_k=tab_k):
          pltpu.async_copy(tab_k[a].at[c_v[s]], rows[b], sem_g[b])

        def g_wait(s, b, a, tab_k=tab_k):
          pltpu.make_async_copy(tab_k[a].at[c_v[s]], rows[b],
                                sem_g[b]).wait()

        def s_start(s, b, a, acc_k=acc_k):
          pltpu.async_copy(rows[b], acc_k[a].at[r_v[s]], sem_s[b],
                           add=True)

        def s_wait(s, b, a, acc_k=acc_k):
          pltpu.make_async_copy(rows[b], acc_k[a].at[r_v[s]],
                                sem_s[b]).wait()

        for a in range(ntab):
          for zz in range(stripe // zr):
            pltpu.sync_copy(
                zbuf, accs[a].at[pl.ds(t * stripe + zz * zr, zr)])
        plsc.subcore_barrier()

        if ntab == 1:
          @pl.when(n_my >= 1)
          def _():
            idx_start(0, 0)

          @pl.when(n_my >= 2)
          def _():
            idx_start(1, 1)

          def pair(i2, s0, s1, n_my=n_my):
            c0 = 2 * i2
            c1 = c0 + 1
            o0 = (s0 + 2) % 4
            o1 = (s1 + 2) % 4
            v1 = c1 < n_my

            @pl.when(i2 >= 1)
            def _():
              s_wait(o0, 0, 0)

              @pl.when(c0 - 1 < n_my)
              def _():
                s_wait(o1, 1, 0)

            idx_wait(c0, s0)
            g_start(s0, 0, 0)

            @pl.when(v1)
            def _():
              idx_wait(c1, s1)
              g_start(s1, 1, 0)

            @pl.when(c0 + 2 < n_my)
            def _():
              idx_start(c0 + 2, o0)

            @pl.when(c1 + 2 < n_my)
            def _():
              idx_start(c1 + 2, o1)

            g_wait(s0, 0, 0)
            s_start(s0, 0, 0)

            @pl.when(v1)
            def _():
              g_wait(s1, 1, 0)
              s_start(s1, 1, 0)

          def pair_body(i2, _):
            @pl.when(i2 % 2 == 0)
            def _():
              pair(i2, 0, 1)

            @pl.when(i2 % 2 == 1)
            def _():
              pair(i2, 2, 3)
            return _

          n_pairs = (n_my + 1) // 2
          lax.fori_loop(0, n_pairs, pair_body, None)
          last = n_pairs - 1

          def drain(s0, s1, n_my=n_my, last=last):
            s_wait(s0, 0, 0)

            @pl.when(n_my % 2 == 0)
            def _():
              s_wait(s1, 1, 0)

          @pl.when((n_my >= 1) & (last % 2 == 0))
          def _():
            drain(0, 1)

          @pl.when((n_my >= 1) & (last % 2 == 1))
          def _():
            drain(2, 3)

        else:
          @pl.when(n_my >= 1)
          def _():
            idx_start(0, 0)

          def single(c, s, o, n_my=n_my):
            @pl.when(c >= 1)
            def _():
              s_wait(o, 0, 0)
              s_wait(o, 1, 1)

            idx_wait(c, s)
            g_start(s, 0, 0)
            g_start(s, 1, 1)

            @pl.when(c + 1 < n_my)
            def _():
              idx_start(c + 1, o)

            g_wait(s, 0, 0)
            s_start(s, 0, 0)
            g_wait(s, 1, 1)
            s_start(s, 1, 1)

          def single_body(c, _):
            @pl.when(c % 2 == 0)
            def _():
              single(c, 0, 1)

            @pl.when(c % 2 == 1)
            def _():
              single(c, 1, 0)
            return _

          lax.fori_loop(0, n_my, single_body, None)
          last = n_my - 1

          @pl.when((n_my >= 1) & (last % 2 == 0))
          def _():
            s_wait(0, 0, 0)
            s_wait(0, 1, 1)

          @pl.when((n_my >= 1) & (last % 2 == 1))
          def _():
            s_wait(1, 0, 0)
            s_wait(1, 1, 1)

        plsc.subcore_barrier()
        for a in range(ntab):
          for off in range(0, stripe, ch):
            sz = min(ch, stripe - off)
            pltpu.sync_copy(accs[a].at[pl.ds(t * stripe + off, sz)],
                            rows[0].at[pl.ds(0, sz)])
            pltpu.sync_copy(
                rows[0].at[pl.ds(0, sz)],
                outs[out_i + a].at[k * 2 + sc, pl.ds(t * stripe + off, sz)])
        plsc.subcore_barrier()
      out_i += ntab

  return pl.kernel(body, out_type=out_type, mesh=_mesh(),
                   scratch_types=scratch, compiler_params=_sc_params())


def _deg_kernel(src, dst, img_r, txt_r):
  n_z = 25
  out_type = [jax.ShapeDtypeStruct((_U,), jnp.float32) for _ in range(4)]
  scratch = [pltpu.VMEM_SHARED((_U,), jnp.float32)] * 2 + [
      pltpu.VMEM((_CHD,), jnp.int32),
      pltpu.VMEM((_CHD,), jnp.float32),
      pltpu.VMEM((2000,), jnp.float32),
  ]

  def body(src_r, dst_r, imr, txr, o_du, o_di, o_dim, o_dtx,
           acc0, acc1, idx_v, ones_v, zb):
    sc = lax.axis_index("c")
    t = lax.axis_index("s")
    _fill(ones_v, _CHD // 16, 1.0)
    _fill(zb, 125, 0.0)

    n_zmy = (n_z - t + _NT - 1) // _NT

    def zero_body(i, _):
      j = i * _NT + t
      pltpu.sync_copy(zb, acc0.at[pl.ds(j * 2000, 2000)])
      pltpu.sync_copy(zb, acc1.at[pl.ds(j * 2000, 2000)])
      return _

    lax.fori_loop(0, n_zmy, zero_body, None)
    plsc.subcore_barrier()

    def count(arr, acc, e):
      nch = e // _CHD
      n_my = (nch - t + _NT - 1) // _NT

      def cb(i, _):
        j = i * _NT + t
        pltpu.sync_copy(arr.at[pl.ds(j * _CHD, _CHD)], idx_v)
        pltpu.sync_copy(ones_v, acc.at[idx_v], add=True)
        return _

      lax.fori_loop(0, n_my, cb, None)

    @pl.when(sc == 0)
    def _():
      count(src_r, acc0, _EUI)
      count(dst_r, acc1, _EUI)

    @pl.when(sc == 1)
    def _():
      count(imr, acc0, _EII)
      count(txr, acc1, _EII)

    plsc.subcore_barrier()

    def wb(acc, out):
      def wbody(i, _):
        j = i * _NT + t
        pltpu.sync_copy(acc.at[pl.ds(j * 2000, 2000)], zb)
        pltpu.sync_copy(zb, out.at[pl.ds(j * 2000, 2000)])
        return _
      lax.fori_loop(0, n_zmy, wbody, None)

    @pl.when(sc == 0)
    def _():
      wb(acc0, o_du)
      wb(acc1, o_di)

    @pl.when(sc == 1)
    def _():
      wb(acc0, o_dim)
      wb(acc1, o_dtx)

  fn = pl.kernel(body, out_type=out_type, mesh=_mesh(),
                 scratch_types=scratch, compiler_params=_sc_params())
  return fn(src, dst, img_r, txt_r)


_B = 1024
_BP = _B // 8


def _zsum_packed(z_ref, nk=4):
  return [z_ref[2 * k] + z_ref[2 * k + 1] for k in range(nk)]


def _pblock(i):
  return (i, 0)


def _full(shape):
  return pl.BlockSpec(shape, lambda i: tuple(0 for _ in shape))


def _pack_scale(v, w=16):
  return jnp.repeat(v.reshape(-1, 128 // w), w, axis=1)


def _scale_packed(chunks, sp, power, m, w=16):
  nk = 64 // w
  mp = m * w // 128
  bp = _B * w // 128

  def body(*refs):
    cs, s_ref, o_refs = refs[:nk], refs[nk], refs[nk + 1:]
    sv = s_ref[...]
    sp_v = sv
    for _ in range(power - 1):
      sp_v = sp_v * sv
    for o, c in zip(o_refs, cs):
      o[...] = sp_v * c[...]

  return pl.pallas_call(
      body,
      grid=(m // _B,),
      in_specs=[pl.BlockSpec((bp, 128), _pblock)] * (nk + 1),
      out_specs=[pl.BlockSpec((bp, 128), _pblock)] * nk,
      out_shape=[jax.ShapeDtypeStruct((mp, 128), jnp.float32)] * nk,
  )(*chunks, sp)


def _scale_z(z, sp, powers, m, w=16):
  nk = 64 // w
  mp = m * w // 128
  bp = _B * w // 128

  def body(z_ref, s_ref, *o_refs):
    zs = _zsum_packed(z_ref, nk)
    sv = s_ref[...]
    oi = 0
    for p in powers:
      sp_v = sv
      for _ in range(p - 1):
        sp_v = sp_v * sv
      for k in range(nk):
        o_refs[oi][...] = sp_v * zs[k]
        oi += 1

  return pl.pallas_call(
      body,
      grid=(m // _B,),
      in_specs=[pl.BlockSpec((2 * nk, bp, 128), lambda i: (0, i, 0)),
                pl.BlockSpec((bp, 128), _pblock)],
      out_specs=[pl.BlockSpec((bp, 128), _pblock)] * (nk * len(powers)),
      out_shape=[jax.ShapeDtypeStruct((mp, 128), jnp.float32)]
      * (nk * len(powers)),
  )(z, sp)


def _feats_kernel(v_feat, t_feat, W_ir, b_ir, W_it, b_it, W_tr, b_tr,
                  W_tt, b_tt):
  def body(v_ref, t_ref, wir, bir, wit, bit, wtr, btr, wtt, btt, oi, ot):
    f = jnp.dot(v_ref[...], wir[...], preferred_element_type=jnp.float32)
    f = jnp.dot(f + bir[...], wit[...], preferred_element_type=jnp.float32)
    oi[...] = jax.nn.sigmoid(f + bit[...])
    g = jnp.dot(t_ref[...], wtr[...], preferred_element_type=jnp.float32)
    g = jnp.dot(g + btr[...], wtt[...], preferred_element_type=jnp.float32)
    ot[...] = jax.nn.sigmoid(g + btt[...])

  return pl.pallas_call(
      body,
      grid=(_IP // _B,),
      in_specs=[pl.BlockSpec((_B, 128), lambda i: (i, 0)),
                pl.BlockSpec((_B, 128), lambda i: (i, 0)),
                _full((128, _D)), _full((1, _D)), _full((_D, _D)),
                _full((1, _D)),
                _full((128, _D)), _full((1, _D)), _full((_D, _D)),
                _full((1, _D))],
      out_specs=[pl.BlockSpec((_B, _D), lambda i: (i, 0))] * 2,
      out_shape=[jax.ShapeDtypeStruct((_IP, _D), jnp.float32)] * 2,
  )(v_feat, t_feat, W_ir, b_ir.reshape(1, -1), W_it, b_it.reshape(1, -1),
    W_tr, b_tr.reshape(1, -1), W_tt, b_tt.reshape(1, -1))


def _final_kernel(user_emb, item_emb, ego1, ego2, img_e_arr, txt_e_arr,
                  W_sc1, b_sc1, w_sc2, W_ib, b_ib, W_tb, b_tb):
  nu = _UP // _B

  def body(ue, ie, e1, e2, ime, txe,
           wsc1, bsc1, wsc2, wib, bib, wtb, btb, o_ref):
    i = pl.program_id(0)
    is_user = i < nu
    ego0 = jnp.where(is_user, ue[...], ie[...])
    content = (ego0 + e1[...] + e2[...]) * (1.0 / 3.0)

    img_e = ime[...]
    txt_e = txe[...]

    si = jnp.dot(jnp.tanh(
        jnp.dot(img_e, wsc1[...], preferred_element_type=jnp.float32)
        + bsc1[...]), wsc2[...], preferred_element_type=jnp.float32)
    st = jnp.dot(jnp.tanh(
        jnp.dot(txt_e, wsc1[...], preferred_element_type=jnp.float32)
        + bsc1[...]), wsc2[...], preferred_element_type=jnp.float32)
    ai = jax.nn.sigmoid(si - st)
    fused = ai * img_e + (1.0 - ai) * txt_e

    sep_i = jax.nn.sigmoid(
        jnp.dot(content, wib[...], preferred_element_type=jnp.float32)
        + bib[...]) * img_e
    sep_t = jax.nn.sigmoid(
        jnp.dot(content, wtb[...], preferred_element_type=jnp.float32)
        + btb[...]) * txt_e
    o_ref[...] = content + fused + sep_i + sep_t

  umap = lambda i: (jnp.minimum(i, nu - 1), 0)
  imap = lambda i: (jnp.maximum(i - nu, 0), 0)
  nspec = pl.BlockSpec((_B, _D), lambda i: (i, 0))
  return pl.pallas_call(
      body,
      grid=(_NP // _B,),
      in_specs=[pl.BlockSpec((_B, _D), umap),
                pl.BlockSpec((_B, _D), imap),
                nspec, nspec, nspec, nspec,
                _full((_D, _D)), _full((1, _D)), _full((_D, 1)),
                _full((_D, _D)), _full((1, _D)),
                _full((_D, _D)), _full((1, _D))],
      out_specs=nspec,
      out_shape=jax.ShapeDtypeStruct((_NP, _D), jnp.float32),
  )(user_emb, item_emb, ego1, ego2, img_e_arr, txt_e_arr,
    W_sc1, b_sc1.reshape(1, -1), w_sc2, W_ib, b_ib.reshape(1, -1),
    W_tb, b_tb.reshape(1, -1))


def _sc_view(packed, m, w=16):
  return packed.reshape(m, w)


def _packed_z(z, m, w=16):
  return z.reshape(z.shape[0], m * w // 128, 128)


def _pack_glue(x, m, w=16):
  return [x[:, w * k:w * (k + 1)].reshape(m * w // 128, 128)
          for k in range(64 // w)]


def _unpack_glue(chunks, m, w=16):
  return jnp.concatenate([c.reshape(m, w) for c in chunks], axis=1)


def _pad_rows(x, rows):
  return jnp.pad(x, ((0, rows - x.shape[0]), (0, 0)))


def _pad_vec(v, n):
  return jnp.pad(v, (0, n - v.shape[0]))


def kernel(user_emb, item_emb, v_feat, t_feat, W_ir, b_ir, W_it, b_it,
           W_tr, b_tr, W_tt, b_tt, W_sc1, b_sc1, w_sc2, W_ib, b_ib,
           W_tb, b_tb, ui_edges, ii_image_edges, ii_text_edges):
  src = ui_edges[0]
  dst = ui_edges[1]

  deg_u, deg_i, d_img, d_txt = _deg_kernel(
      src, dst, ii_image_edges[0], ii_text_edges[0])

  dinv_u = jnp.where(deg_u > 0, lax.rsqrt(deg_u), 0.0)
  dinv_i = jnp.where(deg_i > 0, lax.rsqrt(deg_i), 0.0)
  dp = _pack_scale(jnp.concatenate([_pad_vec(dinv_u, _UP),
                                    _pad_vec(dinv_i, _IP)]))
  dup = _pack_scale(
      _pad_vec(jnp.where(deg_u > 0, 1.0 / deg_u, 0.0), _UP), 32)
  dip = _pack_scale(
      _pad_vec(jnp.where(d_img > 0, lax.rsqrt(d_img), 0.0), _IP), 32)
  dtp = _pack_scale(
      _pad_vec(jnp.where(d_txt > 0, lax.rsqrt(d_txt), 0.0), _IP), 32)

  ue_p = _pad_rows(user_emb, _UP)
  ie_p = _pad_rows(item_emb, _IP)
  vf_p = _pad_rows(v_feat, _IP)
  tf_p = _pad_rows(t_feat, _IP)

  rows = jnp.concatenate([src, dst + _UP])
  cols = jnp.concatenate([dst + _UP, src])

  ue_chunks = _pack_glue(ue_p, _UP)
  ie_chunks = _pack_glue(ie_p, _IP)
  ego0p = [jnp.concatenate([u, i], axis=0)
           for u, i in zip(ue_chunks, ie_chunks)]
  y0p = _scale_packed(ego0p, dp, 1, _NP)

  ui_segsum = _make_segsum(_NP, [(2 * _EUI, 1)], 400)
  (z1,) = ui_segsum(rows, cols, *[_sc_view(y, _NP) for y in y0p])
  sc1 = _scale_z(_packed_z(z1, _NP), dp, (1, 2), _NP)
  ego1p, y1p = sc1[:4], sc1[4:]
  (z2,) = ui_segsum(rows, cols, *[_sc_view(y, _NP) for y in y1p])

  f_img, f_txt = _feats_kernel(vf_p, tf_p, W_ir, b_ir, W_it, b_it,
                               W_tr, b_tr, W_tt, b_tt)
  yi0p = _scale_packed(_pack_glue(f_img, _IP, 32), dip, 1, _IP, 32)
  yt0p = _scale_packed(_pack_glue(f_txt, _IP, 32), dtp, 1, _IP, 32)

  ii_segsum = _make_segsum(_IP, [(_EII, 1), (_EII, 1)], 200, 32)
  zi1, zt1 = ii_segsum(
      ii_image_edges[0], ii_image_edges[1],
      *[_sc_view(y, _IP, 32) for y in yi0p],
      ii_text_edges[0], ii_text_edges[1],
      *[_sc_view(y, _IP, 32) for y in yt0p])
  yi1p = _scale_z(_packed_z(zi1, _IP, 32), dip, (2,), _IP, 32)
  yt1p = _scale_z(_packed_z(zt1, _IP, 32), dtp, (2,), _IP, 32)
  zi2, zt2 = ii_segsum(
      ii_image_edges[0], ii_image_edges[1],
      *[_sc_view(y, _IP, 32) for y in yi1p],
      ii_text_edges[0], ii_text_edges[1],
      *[_sc_view(y, _IP, 32) for y in yt1p])
  img_itp = _scale_z(_packed_z(zi2, _IP, 32), dip, (1,), _IP, 32)
  txt_itp = _scale_z(_packed_z(zt2, _IP, 32), dtp, (1,), _IP, 32)

  uagg = _make_segsum(_UP, [(_EUI, 1)], 200, 32)
  (zu_i,) = uagg(src, dst, *[_sc_view(y, _IP, 32) for y in img_itp])
  (zu_t,) = uagg(src, dst, *[_sc_view(y, _IP, 32) for y in txt_itp])
  imgu_p = _scale_z(_packed_z(zu_i, _UP, 32), dup, (1,), _UP, 32)
  txtu_p = _scale_z(_packed_z(zu_t, _UP, 32), dup, (1,), _UP, 32)

  ego1 = _unpack_glue(ego1p, _NP)
  ego2p = _scale_z(_packed_z(z2, _NP), dp, (1,), _NP)
  ego2 = _unpack_glue(ego2p, _NP)
  img_e = jnp.concatenate(
      [_unpack_glue(imgu_p, _UP, 32), _unpack_glue(img_itp, _IP, 32)],
      axis=0)
  txt_e = jnp.concatenate(
      [_unpack_glue(txtu_p, _UP, 32), _unpack_glue(txt_itp, _IP, 32)],
      axis=0)

  out_pad = _final_kernel(ue_p, ie_p, ego1, ego2, img_e, txt_e,
                          W_sc1, b_sc1, w_sc2, W_ib, b_ib, W_tb, b_tb)
  return jnp.concatenate([out_pad[:_U], out_pad[_UP:_UP + _I]], axis=0)
